# Initial kernel scaffold; baseline (speedup 1.0000x reference)
import jax
import jax.numpy as jnp
from jax.experimental import pallas as pl


def kernel(x, edge_index, W1, att_src1, att_dst1, b1, W2, att_src2, att_dst2, b2):
    raise NotImplementedError("write your pallas kernel here")


# speedup vs baseline: 21.2421x; 21.2421x over previous
import functools

import jax
import jax.numpy as jnp
from jax import lax
from jax.experimental import pallas as pl
from jax.experimental.pallas import tpu as pltpu
from jax.experimental.pallas import tpu_sc as plsc

NN = 10000
EE = 320000
CIN = 128
NH1 = 8
CHID = 64
NQ = 4
QC = 144
NOUT = 40
C2 = 64
NC, NS, LANES = 2, 16, 16
NP = 10240
KCH = 80

_f32 = jnp.float32


def _tc1_body(x_ref, w1_ref, s_ref, d_ref, h1x_ref, abig_ref):
    r = x_ref.shape[0]
    h = jnp.dot(x_ref[...], w1_ref[...], preferred_element_type=_f32)
    ones = jnp.ones((r, 2), _f32)
    zeros = jnp.zeros((r, QC - 130), _f32)
    h1x_ref[0] = jnp.concatenate([h, ones, zeros], axis=1)
    a_s = jnp.dot(h, s_ref[0], preferred_element_type=_f32)
    a_d = jnp.dot(h, d_ref[0], preferred_element_type=_f32)
    abig_ref[0] = jnp.concatenate([a_s, a_d, jnp.zeros((r, 12), _f32)], axis=1)


def _tc1(x, w1, s1, d1):
    rb = 1000
    grid = (NQ, NN // rb)
    return pl.pallas_call(
        _tc1_body,
        grid=grid,
        in_specs=[
            pl.BlockSpec((rb, CIN), lambda q, nb: (nb, 0)),
            pl.BlockSpec((CIN, 128), lambda q, nb: (0, q)),
            pl.BlockSpec((1, 128, 2), lambda q, nb: (q, 0, 0)),
            pl.BlockSpec((1, 128, 2), lambda q, nb: (q, 0, 0)),
        ],
        out_specs=[
            pl.BlockSpec((1, rb, QC), lambda q, nb: (q, nb, 0)),
            pl.BlockSpec((1, rb, 16), lambda q, nb: (q, nb, 0)),
        ],
        out_shape=[
            jax.ShapeDtypeStruct((NQ, NN, QC), _f32),
            jax.ShapeDtypeStruct((NQ, NN, 16), _f32),
        ],
    )(x, w1, s1, d1)


def _sc1_body(src_hbm, dst_hbm, h1x_hbm, abig_hbm, num1_hbm,
              acc, h_rows, av_s, av_d, src_v, dst_v, gidx, didx,
              w0b, w1b, gsem):
    cid = lax.axis_index("c")
    sid = lax.axis_index("s")
    epc = EE // NS
    rpt = NP // NS
    nchunks = epc // KCH

    def zero_acc():
        def zrow(i, _):
            for j in range(QC // 16):
                h_rows[i, pl.ds(j * 16, 16)] = jnp.zeros((16,), _f32)
            return 0
        lax.fori_loop(0, KCH, zrow, 0)
        for z in range(rpt // KCH):
            pltpu.sync_copy(h_rows, acc.at[pl.ds(sid * rpt + z * KCH, KCH)])

    zero_acc()
    plsc.subcore_barrier()

    for p in range(2):
        q = cid * 2 + p

        def chunk(ci, _):
            base = sid * epc + ci * KCH
            pltpu.sync_copy(src_hbm.at[pl.ds(base, KCH)], src_v)
            pltpu.sync_copy(dst_hbm.at[pl.ds(base, KCH)], dst_v)
            for g in range(KCH // LANES):
                sl = pl.ds(g * LANES, LANES)
                gidx[sl] = src_v[sl] + q * NN
                didx[sl] = dst_v[sl] + q * NN
            cp1 = pltpu.async_copy(abig_hbm.at[gidx], av_s, gsem)
            cp2 = pltpu.async_copy(abig_hbm.at[didx], av_d, gsem)
            cp3 = pltpu.async_copy(h1x_hbm.at[gidx], h_rows, gsem)
            cp1.wait()
            cp2.wait()
            for g in range(KCH // LANES):
                sl = pl.ds(g * LANES, LANES)
                lidx = lax.iota(jnp.int32, 16) + g * LANES
                zc = jnp.zeros((16,), jnp.int32)
                x0 = (plsc.load_gather(av_s, [lidx, zc])
                      + plsc.load_gather(av_d, [lidx, zc + 2]))
                x1 = (plsc.load_gather(av_s, [lidx, zc + 1])
                      + plsc.load_gather(av_d, [lidx, zc + 3]))
                w0b[sl] = jnp.exp(jnp.maximum(x0, 0.2 * x0))
                w1b[sl] = jnp.exp(jnp.maximum(x1, 0.2 * x1))
            cp3.wait()

            def edge(e, _):
                ev = jnp.full((16,), e, jnp.int32)
                w0 = plsc.load_gather(w0b, [ev])
                w1 = plsc.load_gather(w1b, [ev])
                for j in range(8):
                    sl = pl.ds(j * 16, 16)
                    w = w0 if j < 4 else w1
                    h_rows[e, sl] = h_rows[e, sl] * w
                lane = lax.iota(jnp.int32, 16)
                wv = jnp.where(lane == 0, w0,
                               jnp.where(lane == 1, w1, jnp.zeros((16,), _f32)))
                h_rows[e, pl.ds(128, 16)] = wv
                return 0
            lax.fori_loop(0, KCH, edge, 0)
            pltpu.sync_copy(h_rows, acc.at[dst_v], add=True)
            return 0
        lax.fori_loop(0, nchunks, chunk, 0)
        plsc.subcore_barrier()

        for z in range(rpt // KCH):
            r0 = sid * rpt + z * KCH
            pltpu.sync_copy(acc.at[pl.ds(r0, KCH)],
                            num1_hbm.at[pl.ds(q * NP + r0, KCH)])
        plsc.subcore_barrier()
        if p == 0:
            zero_acc()
            plsc.subcore_barrier()


def _sc1(src, dst, h1x_flat, abig_flat):
    mesh = plsc.VectorSubcoreMesh(core_axis_name="c", subcore_axis_name="s",
                                  num_cores=NC, num_subcores=NS)
    f = pl.kernel(
        _sc1_body,
        out_type=jax.ShapeDtypeStruct((NQ * NP, QC), _f32),
        mesh=mesh,
        scratch_types=[
            pltpu.VMEM_SHARED((NP, QC), _f32),
            pltpu.VMEM((KCH, QC), _f32),
            pltpu.VMEM((KCH, 16), _f32),
            pltpu.VMEM((KCH, 16), _f32),
            pltpu.VMEM((KCH,), jnp.int32),
            pltpu.VMEM((KCH,), jnp.int32),
            pltpu.VMEM((KCH,), jnp.int32),
            pltpu.VMEM((KCH,), jnp.int32),
            pltpu.VMEM((KCH,), _f32),
            pltpu.VMEM((KCH,), _f32),
            pltpu.SemaphoreType.DMA,
        ],
        compiler_params=pltpu.CompilerParams(needs_layout_passes=False, use_tc_tiling_on_sc=False),
    )
    return f(src, dst, h1x_flat, abig_flat)


def _tc2_body(num1_ref, b1_ref, w2_ref, sd2_ref, h2pad_ref, a2_ref):
    r = num1_ref.shape[1]
    blk = num1_ref[...]
    main = blk[:, :, 0:128].reshape(NQ, r, 2, CHID)
    den = blk[:, :, 128:130].reshape(NQ, r, 2, 1)
    t = main / (den + 1e-16) + b1_ref[...].reshape(NQ, 1, 2, CHID)
    t = jnp.where(t > 0, t, jnp.exp(jnp.minimum(t, 0.0)) - 1.0)
    t = t.reshape(NQ, r, 128)
    h2 = jnp.dot(t[0], w2_ref[0], preferred_element_type=_f32)
    for qq in range(1, NQ):
        h2 = h2 + jnp.dot(t[qq], w2_ref[qq], preferred_element_type=_f32)
    a2 = jnp.dot(h2, sd2_ref[...], preferred_element_type=_f32)
    ones = jnp.ones((r, 1), _f32)
    zeros = jnp.zeros((r, C2 - NOUT - 1), _f32)
    h2pad_ref[...] = jnp.concatenate([h2, ones, zeros], axis=1)
    a2_ref[...] = a2


def _tc2(num1, b1x, w2r, sd2):
    rb = 1000
    grid = (NN // rb,)
    return pl.pallas_call(
        _tc2_body,
        grid=grid,
        in_specs=[
            pl.BlockSpec((NQ, rb, QC), lambda nb: (0, nb, 0)),
            pl.BlockSpec((NQ, 128), lambda nb: (0, 0)),
            pl.BlockSpec((NQ, 128, NOUT), lambda nb: (0, 0, 0)),
            pl.BlockSpec((NOUT, 2), lambda nb: (0, 0)),
        ],
        out_specs=[
            pl.BlockSpec((rb, C2), lambda nb: (nb, 0)),
            pl.BlockSpec((rb, 2), lambda nb: (nb, 0)),
        ],
        out_shape=[
            jax.ShapeDtypeStruct((NN, C2), _f32),
            jax.ShapeDtypeStruct((NN, 2), _f32),
        ],
    )(num1, b1x, w2r, sd2)


def _sc2_body(src_hbm, dst_hbm, h2pad_hbm, a2_hbm, num2_hbm,
              acc, atab, h_rows, src_v, dst_v, wb, gsem):
    cid = lax.axis_index("c")
    sid = lax.axis_index("s")
    epc = EE // (NC * NS)
    rpt = NP // NS
    nchunks = epc // KCH

    def zrow(i, _):
        for j in range(C2 // 16):
            h_rows[i, pl.ds(j * 16, 16)] = jnp.zeros((16,), _f32)
        return 0
    lax.fori_loop(0, KCH, zrow, 0)
    for z in range(rpt // KCH):
        pltpu.sync_copy(h_rows, acc.at[pl.ds(sid * rpt + z * KCH, KCH)])
    pltpu.sync_copy(a2_hbm, atab)
    plsc.subcore_barrier()

    def chunk(ci, _):
        base = cid * (EE // NC) + sid * epc + ci * KCH
        pltpu.sync_copy(src_hbm.at[pl.ds(base, KCH)], src_v)
        pltpu.sync_copy(dst_hbm.at[pl.ds(base, KCH)], dst_v)
        for g in range(KCH // LANES):
            sl = pl.ds(g * LANES, LANES)
            sv = src_v[sl]
            dv = dst_v[sl]
            x = (plsc.load_gather(atab, [sv * 2])
                 + plsc.load_gather(atab, [dv * 2 + 1]))
            wb[sl] = jnp.exp(jnp.maximum(x, 0.2 * x))
        pltpu.async_copy(h2pad_hbm.at[src_v], h_rows, gsem).wait()

        def edge(e, _):
            w = plsc.load_gather(wb, [jnp.full((16,), e, jnp.int32)])
            for j in range(C2 // 16):
                sl = pl.ds(j * 16, 16)
                h_rows[e, sl] = h_rows[e, sl] * w
            return 0
        lax.fori_loop(0, KCH, edge, 0)
        pltpu.sync_copy(h_rows, acc.at[dst_v], add=True)
        return 0
    lax.fori_loop(0, nchunks, chunk, 0)
    plsc.subcore_barrier()

    for z in range(rpt // 128):
        r0 = sid * rpt + z * 128
        pltpu.sync_copy(acc.at[pl.ds(r0, 128)],
                        num2_hbm.at[pl.ds(cid * NP + r0, 128)])


def _sc2(src, dst, h2pad, a2_flat):
    mesh = plsc.VectorSubcoreMesh(core_axis_name="c", subcore_axis_name="s",
                                  num_cores=NC, num_subcores=NS)
    f = pl.kernel(
        _sc2_body,
        out_type=jax.ShapeDtypeStruct((NC * NP, C2), _f32),
        mesh=mesh,
        scratch_types=[
            pltpu.VMEM_SHARED((NP, C2), _f32),
            pltpu.VMEM((2 * NN,), _f32),
            pltpu.VMEM((KCH, C2), _f32),
            pltpu.VMEM((KCH,), jnp.int32),
            pltpu.VMEM((KCH,), jnp.int32),
            pltpu.VMEM((KCH,), _f32),
            pltpu.SemaphoreType.DMA,
        ],
        compiler_params=pltpu.CompilerParams(needs_layout_passes=False, use_tc_tiling_on_sc=False),
    )
    return f(src, dst, h2pad, a2_flat)


def _tc3_body(num2_ref, b2_ref, out_ref):
    s = num2_ref[0] + num2_ref[1]
    den = s[:, NOUT:NOUT + 1]
    o = s[:, 0:NOUT] / (den + 1e-16) + b2_ref[...]
    m = jnp.max(o, axis=1, keepdims=True)
    l = o - m
    out_ref[...] = l - jnp.log(jnp.sum(jnp.exp(l), axis=1, keepdims=True))


def _tc3(num2, b2):
    rb = 1000
    return pl.pallas_call(
        _tc3_body,
        grid=(NN // rb,),
        in_specs=[
            pl.BlockSpec((2, rb, C2), lambda nb: (0, nb, 0)),
            pl.BlockSpec((1, NOUT), lambda nb: (0, 0)),
        ],
        out_specs=pl.BlockSpec((rb, NOUT), lambda nb: (nb, 0)),
        out_shape=jax.ShapeDtypeStruct((NN, NOUT), _f32),
    )(num2, b2)


@jax.jit
def kernel(x, edge_index, W1, att_src1, att_dst1, b1,
           W2, att_src2, att_dst2, b2):
    src = edge_index[0]
    dst = edge_index[1]

    def _proj(att):
        ar = att.reshape(NQ, 2, CHID)
        s = jnp.zeros((NQ, 2, CHID, 2), _f32)
        s = s.at[:, 0, :, 0].set(ar[:, 0]).at[:, 1, :, 1].set(ar[:, 1])
        return s.reshape(NQ, 128, 2)

    s1 = _proj(att_src1)
    d1 = _proj(att_dst1)
    b1x = b1.reshape(NQ, 128)
    w2r = W2.reshape(NQ, 128, NOUT)
    sd2 = jnp.stack([att_src2[0], att_dst2[0]], axis=1)

    h1x, abig = _tc1(x, W1, s1, d1)
    num1 = _sc1(src, dst, h1x.reshape(NQ * NN, QC),
                abig.reshape(NQ * NN, 16))
    h2pad, a2 = _tc2(num1.reshape(NQ, NP, QC), b1x, w2r, sd2)
    num2 = _sc2(src, dst, h2pad, a2.reshape(-1))
    return _tc3(num2.reshape(2, NP, C2), b2.reshape(1, NOUT))


# speedup vs baseline: 32.6258x; 1.5359x over previous
import functools

import jax
import jax.numpy as jnp
from jax import lax
from jax.experimental import pallas as pl
from jax.experimental.pallas import tpu as pltpu
from jax.experimental.pallas import tpu_sc as plsc

NN = 10000
EE = 320000
CIN = 128
NH1 = 8
CHID = 64
NQ = 4
QC = 144
NOUT = 40
C2 = 64
NC, NS, LANES = 2, 16, 16
NP = 10240
KCH = 80

_f32 = jnp.float32


def _tc1_body(x_ref, w1_ref, s_ref, d_ref, h1x_ref, abig_ref):
    r = x_ref.shape[0]
    h = jnp.dot(x_ref[...], w1_ref[...], preferred_element_type=_f32)
    ones = jnp.ones((r, 2), _f32)
    zeros = jnp.zeros((r, QC - 130), _f32)
    h1x_ref[0] = jnp.concatenate([h, ones, zeros], axis=1)
    a_s = jnp.dot(h, s_ref[0], preferred_element_type=_f32)
    a_d = jnp.dot(h, d_ref[0], preferred_element_type=_f32)
    abig_ref[0] = jnp.concatenate([a_s, a_d, jnp.zeros((r, 12), _f32)], axis=1)


def _tc1(x, w1, s1, d1):
    rb = 1000
    grid = (NQ, NN // rb)
    return pl.pallas_call(
        _tc1_body,
        grid=grid,
        in_specs=[
            pl.BlockSpec((rb, CIN), lambda q, nb: (nb, 0)),
            pl.BlockSpec((CIN, 128), lambda q, nb: (0, q)),
            pl.BlockSpec((1, 128, 2), lambda q, nb: (q, 0, 0)),
            pl.BlockSpec((1, 128, 2), lambda q, nb: (q, 0, 0)),
        ],
        out_specs=[
            pl.BlockSpec((1, rb, QC), lambda q, nb: (q, nb, 0)),
            pl.BlockSpec((1, rb, 16), lambda q, nb: (q, nb, 0)),
        ],
        out_shape=[
            jax.ShapeDtypeStruct((NQ, NN, QC), _f32),
            jax.ShapeDtypeStruct((NQ, NN, 16), _f32),
        ],
    )(x, w1, s1, d1)


def _sc1_body(src_hbm, dst_hbm, h1x_hbm, abig_hbm, num1_hbm,
              acc,
              h_rows0, av_s0, av_d0, src_v0, dst_v0, gidx0, didx0, w0b0, w1b0,
              h_rows1, av_s1, av_d1, src_v1, dst_v1, gidx1, didx1, w0b1, w1b1,
              gsem0, gsem1, ssem0, ssem1):
    cid = lax.axis_index("c")
    sid = lax.axis_index("s")
    epc = EE // NS
    rpt = NP // NS
    nchunks = epc // KCH
    npairs = nchunks // 2

    bufs0 = (h_rows0, av_s0, av_d0, src_v0, dst_v0, gidx0, didx0, w0b0, w1b0,
             gsem0, ssem0)
    bufs1 = (h_rows1, av_s1, av_d1, src_v1, dst_v1, gidx1, didx1, w0b1, w1b1,
             gsem1, ssem1)

    def zero_acc():
        def zrow(i, _):
            for j in range(QC // 16):
                h_rows0[i, pl.ds(j * 16, 16)] = jnp.zeros((16,), _f32)
            return 0
        lax.fori_loop(0, KCH, zrow, 0)
        for z in range(rpt // KCH):
            pltpu.sync_copy(h_rows0, acc.at[pl.ds(sid * rpt + z * KCH, KCH)])

    def loadidx(q, base, b):
        h_rows, av_s, av_d, src_v, dst_v, gidx, didx, w0b, w1b, gsem, ssem = b
        pltpu.sync_copy(src_hbm.at[pl.ds(base, KCH)], src_v)
        pltpu.sync_copy(dst_hbm.at[pl.ds(base, KCH)], dst_v)
        for g in range(KCH // LANES):
            sl = pl.ds(g * LANES, LANES)
            gidx[sl] = src_v[sl] + q * NN
            didx[sl] = dst_v[sl] + q * NN
        pltpu.async_copy(abig_hbm.at[gidx], av_s, gsem)
        pltpu.async_copy(abig_hbm.at[didx], av_d, gsem)
        pltpu.async_copy(h1x_hbm.at[gidx], h_rows, gsem)

    def process(b):
        h_rows, av_s, av_d, src_v, dst_v, gidx, didx, w0b, w1b, gsem, ssem = b
        pltpu.make_async_copy(abig_hbm.at[gidx], av_s, gsem).wait()
        pltpu.make_async_copy(abig_hbm.at[didx], av_d, gsem).wait()
        for g in range(KCH // LANES):
            sl = pl.ds(g * LANES, LANES)
            lidx = lax.iota(jnp.int32, 16) + g * LANES
            zc = jnp.zeros((16,), jnp.int32)
            x0 = (plsc.load_gather(av_s, [lidx, zc])
                  + plsc.load_gather(av_d, [lidx, zc + 2]))
            x1 = (plsc.load_gather(av_s, [lidx, zc + 1])
                  + plsc.load_gather(av_d, [lidx, zc + 3]))
            w0b[sl] = jnp.exp(jnp.maximum(x0, 0.2 * x0))
            w1b[sl] = jnp.exp(jnp.maximum(x1, 0.2 * x1))
        pltpu.make_async_copy(h1x_hbm.at[gidx], h_rows, gsem).wait()

        def edge(e, _):
            ev = jnp.full((16,), e, jnp.int32)
            w0 = plsc.load_gather(w0b, [ev])
            w1 = plsc.load_gather(w1b, [ev])
            for j in range(8):
                sl = pl.ds(j * 16, 16)
                w = w0 if j < 4 else w1
                h_rows[e, sl] = h_rows[e, sl] * w
            lane = lax.iota(jnp.int32, 16)
            wv = jnp.where(lane == 0, w0,
                           jnp.where(lane == 1, w1, jnp.zeros((16,), _f32)))
            h_rows[e, pl.ds(128, 16)] = wv
            return 0
        lax.fori_loop(0, KCH, edge, 0, unroll=2)
        pltpu.async_copy(h_rows, acc.at[dst_v], ssem, add=True)

    def wait_scat(b):
        h_rows, av_s, av_d, src_v, dst_v, gidx, didx, w0b, w1b, gsem, ssem = b
        pltpu.make_async_copy(h_rows, acc.at[dst_v], ssem).wait()

    zero_acc()
    plsc.subcore_barrier()

    for p in range(2):
        q = cid * 2 + p
        sbase = sid * epc
        loadidx(q, sbase, bufs0)

        def pair(i, _):
            c0 = i * 2

            @pl.when(i > 0)
            def _():
                wait_scat(bufs1)
            loadidx(q, sbase + (c0 + 1) * KCH, bufs1)
            process(bufs0)
            process(bufs1)
            wait_scat(bufs0)

            @pl.when(i < npairs - 1)
            def _():
                loadidx(q, sbase + (c0 + 2) * KCH, bufs0)
            return 0
        lax.fori_loop(0, npairs, pair, 0)
        wait_scat(bufs1)
        plsc.subcore_barrier()

        for z in range(rpt // KCH):
            r0 = sid * rpt + z * KCH
            pltpu.sync_copy(acc.at[pl.ds(r0, KCH)],
                            num1_hbm.at[pl.ds(q * NP + r0, KCH)])
        plsc.subcore_barrier()
        if p == 0:
            zero_acc()
            plsc.subcore_barrier()


def _sc1(src, dst, h1x_flat, abig_flat):
    mesh = plsc.VectorSubcoreMesh(core_axis_name="c", subcore_axis_name="s",
                                  num_cores=NC, num_subcores=NS)
    f = pl.kernel(
        _sc1_body,
        out_type=jax.ShapeDtypeStruct((NQ * NP, QC), _f32),
        mesh=mesh,
        scratch_types=(
            [pltpu.VMEM_SHARED((NP, QC), _f32)]
            + 2 * [pltpu.VMEM((KCH, QC), _f32),
                   pltpu.VMEM((KCH, 16), _f32),
                   pltpu.VMEM((KCH, 16), _f32),
                   pltpu.VMEM((KCH,), jnp.int32),
                   pltpu.VMEM((KCH,), jnp.int32),
                   pltpu.VMEM((KCH,), jnp.int32),
                   pltpu.VMEM((KCH,), jnp.int32),
                   pltpu.VMEM((KCH,), _f32),
                   pltpu.VMEM((KCH,), _f32)]
            + 4 * [pltpu.SemaphoreType.DMA]
        ),
        compiler_params=pltpu.CompilerParams(needs_layout_passes=False, use_tc_tiling_on_sc=False),
    )
    return f(src, dst, h1x_flat, abig_flat)


def _tc2_body(num1_ref, b1_ref, w2_ref, sd2_ref, h2pad_ref, a2_ref):
    r = num1_ref.shape[1]
    blk = num1_ref[...]
    main = blk[:, :, 0:128].reshape(NQ, r, 2, CHID)
    den = blk[:, :, 128:130].reshape(NQ, r, 2, 1)
    t = main / (den + 1e-16) + b1_ref[...].reshape(NQ, 1, 2, CHID)
    t = jnp.where(t > 0, t, jnp.exp(jnp.minimum(t, 0.0)) - 1.0)
    t = t.reshape(NQ, r, 128)
    h2 = jnp.dot(t[0], w2_ref[0], preferred_element_type=_f32)
    for qq in range(1, NQ):
        h2 = h2 + jnp.dot(t[qq], w2_ref[qq], preferred_element_type=_f32)
    a2 = jnp.dot(h2, sd2_ref[...], preferred_element_type=_f32)
    ones = jnp.ones((r, 1), _f32)
    zeros = jnp.zeros((r, C2 - NOUT - 1), _f32)
    h2pad_ref[...] = jnp.concatenate([h2, ones, zeros], axis=1)
    a2_ref[...] = a2


def _tc2(num1, b1x, w2r, sd2):
    rb = 1000
    grid = (NN // rb,)
    return pl.pallas_call(
        _tc2_body,
        grid=grid,
        in_specs=[
            pl.BlockSpec((NQ, rb, QC), lambda nb: (0, nb, 0)),
            pl.BlockSpec((NQ, 128), lambda nb: (0, 0)),
            pl.BlockSpec((NQ, 128, NOUT), lambda nb: (0, 0, 0)),
            pl.BlockSpec((NOUT, 2), lambda nb: (0, 0)),
        ],
        out_specs=[
            pl.BlockSpec((rb, C2), lambda nb: (nb, 0)),
            pl.BlockSpec((rb, 2), lambda nb: (nb, 0)),
        ],
        out_shape=[
            jax.ShapeDtypeStruct((NN, C2), _f32),
            jax.ShapeDtypeStruct((NN, 2), _f32),
        ],
    )(num1, b1x, w2r, sd2)


def _sc2_body(src_hbm, dst_hbm, h2pad_hbm, a2_hbm, num2_hbm,
              acc, atab,
              h_rows0, src_v0, dst_v0, wb0,
              h_rows1, src_v1, dst_v1, wb1,
              gsem0, gsem1, ssem0, ssem1):
    cid = lax.axis_index("c")
    sid = lax.axis_index("s")
    epc = EE // (NC * NS)
    rpt = NP // NS
    nchunks = epc // KCH
    npairs = nchunks // 2

    bufs0 = (h_rows0, src_v0, dst_v0, wb0, gsem0, ssem0)
    bufs1 = (h_rows1, src_v1, dst_v1, wb1, gsem1, ssem1)

    def zrow(i, _):
        for j in range(C2 // 16):
            h_rows0[i, pl.ds(j * 16, 16)] = jnp.zeros((16,), _f32)
        return 0
    lax.fori_loop(0, KCH, zrow, 0)
    for z in range(rpt // KCH):
        pltpu.sync_copy(h_rows0, acc.at[pl.ds(sid * rpt + z * KCH, KCH)])
    pltpu.sync_copy(a2_hbm, atab)
    plsc.subcore_barrier()

    ebase = cid * (EE // NC) + sid * epc

    def loadidx(base, b):
        h_rows, src_v, dst_v, wb, gsem, ssem = b
        pltpu.sync_copy(src_hbm.at[pl.ds(base, KCH)], src_v)
        pltpu.sync_copy(dst_hbm.at[pl.ds(base, KCH)], dst_v)
        pltpu.async_copy(h2pad_hbm.at[src_v], h_rows, gsem)

    def process(b):
        h_rows, src_v, dst_v, wb, gsem, ssem = b
        for g in range(KCH // LANES):
            sl = pl.ds(g * LANES, LANES)
            sv = src_v[sl]
            dv = dst_v[sl]
            x = (plsc.load_gather(atab, [sv * 2])
                 + plsc.load_gather(atab, [dv * 2 + 1]))
            wb[sl] = jnp.exp(jnp.maximum(x, 0.2 * x))
        pltpu.make_async_copy(h2pad_hbm.at[src_v], h_rows, gsem).wait()

        def edge(e, _):
            w = plsc.load_gather(wb, [jnp.full((16,), e, jnp.int32)])
            for j in range(C2 // 16):
                sl = pl.ds(j * 16, 16)
                h_rows[e, sl] = h_rows[e, sl] * w
            return 0
        lax.fori_loop(0, KCH, edge, 0, unroll=4)
        pltpu.async_copy(h_rows, acc.at[dst_v], ssem, add=True)

    def wait_scat(b):
        h_rows, src_v, dst_v, wb, gsem, ssem = b
        pltpu.make_async_copy(h_rows, acc.at[dst_v], ssem).wait()

    loadidx(ebase, bufs0)

    def pair(i, _):
        c0 = i * 2

        @pl.when(i > 0)
        def _():
            wait_scat(bufs1)
        loadidx(ebase + (c0 + 1) * KCH, bufs1)
        process(bufs0)
        process(bufs1)
        wait_scat(bufs0)

        @pl.when(i < npairs - 1)
        def _():
            loadidx(ebase + (c0 + 2) * KCH, bufs0)
        return 0
    lax.fori_loop(0, npairs, pair, 0)
    wait_scat(bufs1)
    plsc.subcore_barrier()

    for z in range(rpt // KCH):
        r0 = sid * rpt + z * KCH
        pltpu.sync_copy(acc.at[pl.ds(r0, KCH)],
                        num2_hbm.at[pl.ds(cid * NP + r0, KCH)])


def _sc2(src, dst, h2pad, a2_flat):
    mesh = plsc.VectorSubcoreMesh(core_axis_name="c", subcore_axis_name="s",
                                  num_cores=NC, num_subcores=NS)
    f = pl.kernel(
        _sc2_body,
        out_type=jax.ShapeDtypeStruct((NC * NP, C2), _f32),
        mesh=mesh,
        scratch_types=(
            [pltpu.VMEM_SHARED((NP, C2), _f32),
             pltpu.VMEM((2 * NN,), _f32)]
            + 2 * [pltpu.VMEM((KCH, C2), _f32),
                   pltpu.VMEM((KCH,), jnp.int32),
                   pltpu.VMEM((KCH,), jnp.int32),
                   pltpu.VMEM((KCH,), _f32)]
            + 4 * [pltpu.SemaphoreType.DMA]
        ),
        compiler_params=pltpu.CompilerParams(needs_layout_passes=False, use_tc_tiling_on_sc=False),
    )
    return f(src, dst, h2pad, a2_flat)


def _tc3_body(num2_ref, b2_ref, out_ref):
    s = num2_ref[0] + num2_ref[1]
    den = s[:, NOUT:NOUT + 1]
    o = s[:, 0:NOUT] / (den + 1e-16) + b2_ref[...]
    m = jnp.max(o, axis=1, keepdims=True)
    l = o - m
    out_ref[...] = l - jnp.log(jnp.sum(jnp.exp(l), axis=1, keepdims=True))


def _tc3(num2, b2):
    rb = 1000
    return pl.pallas_call(
        _tc3_body,
        grid=(NN // rb,),
        in_specs=[
            pl.BlockSpec((2, rb, C2), lambda nb: (0, nb, 0)),
            pl.BlockSpec((1, NOUT), lambda nb: (0, 0)),
        ],
        out_specs=pl.BlockSpec((rb, NOUT), lambda nb: (nb, 0)),
        out_shape=jax.ShapeDtypeStruct((NN, NOUT), _f32),
    )(num2, b2)


@jax.jit
def kernel(x, edge_index, W1, att_src1, att_dst1, b1,
           W2, att_src2, att_dst2, b2):
    src = edge_index[0]
    dst = edge_index[1]

    def _proj(att):
        ar = att.reshape(NQ, 2, CHID)
        s = jnp.zeros((NQ, 2, CHID, 2), _f32)
        s = s.at[:, 0, :, 0].set(ar[:, 0]).at[:, 1, :, 1].set(ar[:, 1])
        return s.reshape(NQ, 128, 2)

    s1 = _proj(att_src1)
    d1 = _proj(att_dst1)
    b1x = b1.reshape(NQ, 128)
    w2r = W2.reshape(NQ, 128, NOUT)
    sd2 = jnp.stack([att_src2[0], att_dst2[0]], axis=1)

    h1x, abig = _tc1(x, W1, s1, d1)
    num1 = _sc1(src, dst, h1x.reshape(NQ * NN, QC),
                abig.reshape(NQ * NN, 16))
    h2pad, a2 = _tc2(num1.reshape(NQ, NP, QC), b1x, w2r, sd2)
    num2 = _sc2(src, dst, h2pad, a2.reshape(-1))
    return _tc3(num2.reshape(2, NP, C2), b2.reshape(1, NOUT))


# speedup vs baseline: 33.0858x; 1.0141x over previous
import functools

import jax
import jax.numpy as jnp
from jax import lax
from jax.experimental import pallas as pl
from jax.experimental.pallas import tpu as pltpu
from jax.experimental.pallas import tpu_sc as plsc

NN = 10000
EE = 320000
CIN = 128
NH1 = 8
CHID = 64
NQ = 4
QC = 144
NOUT = 40
C2 = 64
NC, NS, LANES = 2, 16, 16
NP = 10240
KCH = 80
KC2 = 80

_f32 = jnp.float32


def _tc1_body(x_ref, w1_ref, s_ref, d_ref, h1x_ref, abig_ref):
    r = x_ref.shape[0]
    h = jnp.dot(x_ref[...], w1_ref[...], preferred_element_type=_f32)
    ones = jnp.ones((r, 2), _f32)
    zeros = jnp.zeros((r, QC - 130), _f32)
    h1x_ref[0] = jnp.concatenate([h, ones, zeros], axis=1)
    a_s = jnp.dot(h, s_ref[0], preferred_element_type=_f32)
    a_d = jnp.dot(h, d_ref[0], preferred_element_type=_f32)
    abig_ref[0] = jnp.concatenate([a_s, a_d, jnp.zeros((r, 12), _f32)], axis=1)


def _tc1(x, w1, s1, d1):
    rb = 1000
    grid = (NQ, NN // rb)
    return pl.pallas_call(
        _tc1_body,
        grid=grid,
        in_specs=[
            pl.BlockSpec((rb, CIN), lambda q, nb: (nb, 0)),
            pl.BlockSpec((CIN, 128), lambda q, nb: (0, q)),
            pl.BlockSpec((1, 128, 2), lambda q, nb: (q, 0, 0)),
            pl.BlockSpec((1, 128, 2), lambda q, nb: (q, 0, 0)),
        ],
        out_specs=[
            pl.BlockSpec((1, rb, QC), lambda q, nb: (q, nb, 0)),
            pl.BlockSpec((1, rb, 16), lambda q, nb: (q, nb, 0)),
        ],
        out_shape=[
            jax.ShapeDtypeStruct((NQ, NN, QC), _f32),
            jax.ShapeDtypeStruct((NQ, NN, 16), _f32),
        ],
    )(x, w1, s1, d1)


def _sc1_body(src_hbm, dst_hbm, h1x_hbm, abig_hbm, num1_hbm,
              acc,
              h_rows0, av_s0, av_d0, src_v0, dst_v0, gidx0, didx0, w0b0, w1b0,
              h_rows1, av_s1, av_d1, src_v1, dst_v1, gidx1, didx1, w0b1, w1b1,
              gsem0, asem0, bsem0, ssem0, gsem1, asem1, bsem1, ssem1):
    cid = lax.axis_index("c")
    sid = lax.axis_index("s")
    epc = EE // NS
    rpt = NP // NS
    nchunks = epc // KCH
    npairs = nchunks // 2

    bufs0 = (h_rows0, av_s0, av_d0, src_v0, dst_v0, gidx0, didx0, w0b0, w1b0,
             gsem0, asem0, bsem0, ssem0)
    bufs1 = (h_rows1, av_s1, av_d1, src_v1, dst_v1, gidx1, didx1, w0b1, w1b1,
             gsem1, asem1, bsem1, ssem1)

    def zero_acc():
        def zrow(i, _):
            for j in range(QC // 16):
                h_rows0[i, pl.ds(j * 16, 16)] = jnp.zeros((16,), _f32)
            return 0
        lax.fori_loop(0, KCH, zrow, 0)
        for z in range(rpt // KCH):
            pltpu.sync_copy(h_rows0, acc.at[pl.ds(sid * rpt + z * KCH, KCH)])

    def loadidx(q, base, b):
        (h_rows, av_s, av_d, src_v, dst_v, gidx, didx, w0b, w1b,
         gsem, asem, bsem, ssem) = b
        pltpu.sync_copy(src_hbm.at[pl.ds(base, KCH)], src_v)
        pltpu.sync_copy(dst_hbm.at[pl.ds(base, KCH)], dst_v)
        for g in range(KCH // LANES):
            sl = pl.ds(g * LANES, LANES)
            gidx[sl] = src_v[sl] + q * NN
            didx[sl] = dst_v[sl] + q * NN
        pltpu.async_copy(abig_hbm.at[gidx], av_s, asem)
        pltpu.async_copy(abig_hbm.at[didx], av_d, bsem)
        pltpu.async_copy(h1x_hbm.at[gidx], h_rows, gsem)

    def process(b):
        (h_rows, av_s, av_d, src_v, dst_v, gidx, didx, w0b, w1b,
         gsem, asem, bsem, ssem) = b
        pltpu.make_async_copy(abig_hbm.at[gidx], av_s, asem).wait()
        pltpu.make_async_copy(abig_hbm.at[didx], av_d, bsem).wait()
        for g in range(KCH // LANES):
            sl = pl.ds(g * LANES, LANES)
            lidx = lax.iota(jnp.int32, 16) + g * LANES
            zc = jnp.zeros((16,), jnp.int32)
            x0 = (plsc.load_gather(av_s, [lidx, zc])
                  + plsc.load_gather(av_d, [lidx, zc + 2]))
            x1 = (plsc.load_gather(av_s, [lidx, zc + 1])
                  + plsc.load_gather(av_d, [lidx, zc + 3]))
            w0b[sl] = jnp.exp(jnp.maximum(x0, 0.2 * x0))
            w1b[sl] = jnp.exp(jnp.maximum(x1, 0.2 * x1))
        pltpu.make_async_copy(h1x_hbm.at[gidx], h_rows, gsem).wait()

        def edge(e, _):
            ev = jnp.full((16,), e, jnp.int32)
            w0 = plsc.load_gather(w0b, [ev])
            w1 = plsc.load_gather(w1b, [ev])
            for j in range(8):
                sl = pl.ds(j * 16, 16)
                w = w0 if j < 4 else w1
                h_rows[e, sl] = h_rows[e, sl] * w
            lane = lax.iota(jnp.int32, 16)
            wv = jnp.where(lane == 0, w0,
                           jnp.where(lane == 1, w1, jnp.zeros((16,), _f32)))
            h_rows[e, pl.ds(128, 16)] = wv
            return 0
        lax.fori_loop(0, KCH, edge, 0, unroll=2)
        pltpu.async_copy(h_rows, acc.at[dst_v], ssem, add=True)

    def wait_scat(b):
        (h_rows, av_s, av_d, src_v, dst_v, gidx, didx, w0b, w1b,
         gsem, asem, bsem, ssem) = b
        pltpu.make_async_copy(h_rows, acc.at[dst_v], ssem).wait()

    zero_acc()
    plsc.subcore_barrier()

    for p in range(2):
        q = cid * 2 + p
        sbase = sid * epc
        loadidx(q, sbase, bufs0)

        def pair(i, _):
            c0 = i * 2

            @pl.when(i > 0)
            def _():
                wait_scat(bufs1)
            loadidx(q, sbase + (c0 + 1) * KCH, bufs1)
            process(bufs0)
            process(bufs1)
            wait_scat(bufs0)

            @pl.when(i < npairs - 1)
            def _():
                loadidx(q, sbase + (c0 + 2) * KCH, bufs0)
            return 0
        lax.fori_loop(0, npairs, pair, 0)
        wait_scat(bufs1)
        plsc.subcore_barrier()

        for z in range(rpt // KCH):
            r0 = sid * rpt + z * KCH
            pltpu.sync_copy(acc.at[pl.ds(r0, KCH)],
                            num1_hbm.at[pl.ds(q * NP + r0, KCH)])
        plsc.subcore_barrier()
        if p == 0:
            zero_acc()
            plsc.subcore_barrier()


def _sc1(src, dst, h1x_flat, abig_flat):
    mesh = plsc.VectorSubcoreMesh(core_axis_name="c", subcore_axis_name="s",
                                  num_cores=NC, num_subcores=NS)
    f = pl.kernel(
        _sc1_body,
        out_type=jax.ShapeDtypeStruct((NQ * NP, QC), _f32),
        mesh=mesh,
        scratch_types=(
            [pltpu.VMEM_SHARED((NP, QC), _f32)]
            + 2 * [pltpu.VMEM((KCH, QC), _f32),
                   pltpu.VMEM((KCH, 16), _f32),
                   pltpu.VMEM((KCH, 16), _f32),
                   pltpu.VMEM((KCH,), jnp.int32),
                   pltpu.VMEM((KCH,), jnp.int32),
                   pltpu.VMEM((KCH,), jnp.int32),
                   pltpu.VMEM((KCH,), jnp.int32),
                   pltpu.VMEM((KCH,), _f32),
                   pltpu.VMEM((KCH,), _f32)]
            + 8 * [pltpu.SemaphoreType.DMA]
        ),
        compiler_params=pltpu.CompilerParams(needs_layout_passes=False, use_tc_tiling_on_sc=False),
    )
    return f(src, dst, h1x_flat, abig_flat)


def _tc2_body(num1_ref, b1_ref, w2_ref, sd2_ref, h2pad_ref, a2_ref):
    r = num1_ref.shape[1]
    blk = num1_ref[...]
    main = blk[:, :, 0:128].reshape(NQ, r, 2, CHID)
    den = blk[:, :, 128:130].reshape(NQ, r, 2, 1)
    t = main / (den + 1e-16) + b1_ref[...].reshape(NQ, 1, 2, CHID)
    t = jnp.where(t > 0, t, jnp.exp(jnp.minimum(t, 0.0)) - 1.0)
    t = t.reshape(NQ, r, 128)
    h2 = jnp.dot(t[0], w2_ref[0], preferred_element_type=_f32)
    for qq in range(1, NQ):
        h2 = h2 + jnp.dot(t[qq], w2_ref[qq], preferred_element_type=_f32)
    a2 = jnp.dot(h2, sd2_ref[...], preferred_element_type=_f32)
    ones = jnp.ones((r, 1), _f32)
    zeros = jnp.zeros((r, C2 - NOUT - 1), _f32)
    h2pad_ref[...] = jnp.concatenate([h2, ones, zeros], axis=1)
    a2_ref[...] = a2


def _tc2(num1, b1x, w2r, sd2):
    rb = 1000
    grid = (NN // rb,)
    return pl.pallas_call(
        _tc2_body,
        grid=grid,
        in_specs=[
            pl.BlockSpec((NQ, rb, QC), lambda nb: (0, nb, 0)),
            pl.BlockSpec((NQ, 128), lambda nb: (0, 0)),
            pl.BlockSpec((NQ, 128, NOUT), lambda nb: (0, 0, 0)),
            pl.BlockSpec((NOUT, 2), lambda nb: (0, 0)),
        ],
        out_specs=[
            pl.BlockSpec((rb, C2), lambda nb: (nb, 0)),
            pl.BlockSpec((rb, 2), lambda nb: (nb, 0)),
        ],
        out_shape=[
            jax.ShapeDtypeStruct((NN, C2), _f32),
            jax.ShapeDtypeStruct((NN, 2), _f32),
        ],
    )(num1, b1x, w2r, sd2)


def _sc2_body(src_hbm, dst_hbm, h2pad_hbm, a2_hbm, num2_hbm,
              acc, atab,
              h_rows0, src_v0, dst_v0, wb0,
              h_rows1, src_v1, dst_v1, wb1,
              gsem0, gsem1, ssem0, ssem1):
    cid = lax.axis_index("c")
    sid = lax.axis_index("s")
    epc = EE // (NC * NS)
    rpt = NP // NS
    nchunks = epc // KC2
    npairs = nchunks // 2

    bufs0 = (h_rows0, src_v0, dst_v0, wb0, gsem0, ssem0)
    bufs1 = (h_rows1, src_v1, dst_v1, wb1, gsem1, ssem1)

    def zrow(i, _):
        for j in range(C2 // 16):
            h_rows0[i, pl.ds(j * 16, 16)] = jnp.zeros((16,), _f32)
        return 0
    lax.fori_loop(0, 80, zrow, 0)
    for z in range(rpt // 80):
        pltpu.sync_copy(h_rows0.at[pl.ds(0, 80)],
                        acc.at[pl.ds(sid * rpt + z * 80, 80)])
    pltpu.sync_copy(a2_hbm, atab)
    plsc.subcore_barrier()

    ebase = cid * (EE // NC) + sid * epc

    def loadidx(base, b):
        h_rows, src_v, dst_v, wb, gsem, ssem = b
        pltpu.sync_copy(src_hbm.at[pl.ds(base, KC2)], src_v)
        pltpu.sync_copy(dst_hbm.at[pl.ds(base, KC2)], dst_v)
        pltpu.async_copy(h2pad_hbm.at[src_v], h_rows, gsem)

    def process(b):
        h_rows, src_v, dst_v, wb, gsem, ssem = b
        for g in range(KC2 // LANES):
            sl = pl.ds(g * LANES, LANES)
            sv = src_v[sl]
            dv = dst_v[sl]
            x = (plsc.load_gather(atab, [sv * 2])
                 + plsc.load_gather(atab, [dv * 2 + 1]))
            wb[sl] = jnp.exp(jnp.maximum(x, 0.2 * x))
        pltpu.make_async_copy(h2pad_hbm.at[src_v], h_rows, gsem).wait()

        def edge(e, _):
            w = plsc.load_gather(wb, [jnp.full((16,), e, jnp.int32)])
            for j in range(C2 // 16):
                sl = pl.ds(j * 16, 16)
                h_rows[e, sl] = h_rows[e, sl] * w
            return 0
        lax.fori_loop(0, KC2, edge, 0, unroll=4)
        pltpu.async_copy(h_rows, acc.at[dst_v], ssem, add=True)

    def wait_scat(b):
        h_rows, src_v, dst_v, wb, gsem, ssem = b
        pltpu.make_async_copy(h_rows, acc.at[dst_v], ssem).wait()

    loadidx(ebase, bufs0)

    def pair(i, _):
        c0 = i * 2

        @pl.when(i > 0)
        def _():
            wait_scat(bufs1)
        loadidx(ebase + (c0 + 1) * KC2, bufs1)
        process(bufs0)
        process(bufs1)
        wait_scat(bufs0)

        loadidx(ebase + (c0 + 2) * KC2, bufs0)
        return 0
    lax.fori_loop(0, npairs, pair, 0)
    process(bufs0)
    wait_scat(bufs0)
    wait_scat(bufs1)
    plsc.subcore_barrier()

    for z in range(rpt // 80):
        r0 = sid * rpt + z * 80
        pltpu.sync_copy(acc.at[pl.ds(r0, 80)],
                        num2_hbm.at[pl.ds(cid * NP + r0, 80)])


def _sc2(src, dst, h2pad, a2_flat):
    mesh = plsc.VectorSubcoreMesh(core_axis_name="c", subcore_axis_name="s",
                                  num_cores=NC, num_subcores=NS)
    f = pl.kernel(
        _sc2_body,
        out_type=jax.ShapeDtypeStruct((NC * NP, C2), _f32),
        mesh=mesh,
        scratch_types=(
            [pltpu.VMEM_SHARED((NP, C2), _f32),
             pltpu.VMEM((2 * NN,), _f32)]
            + 2 * [pltpu.VMEM((KC2, C2), _f32),
                   pltpu.VMEM((KC2,), jnp.int32),
                   pltpu.VMEM((KC2,), jnp.int32),
                   pltpu.VMEM((KC2,), _f32)]
            + 4 * [pltpu.SemaphoreType.DMA]
        ),
        compiler_params=pltpu.CompilerParams(needs_layout_passes=False, use_tc_tiling_on_sc=False),
    )
    return f(src, dst, h2pad, a2_flat)


def _tc3_body(num2_ref, b2_ref, out_ref):
    s = num2_ref[0] + num2_ref[1]
    den = s[:, NOUT:NOUT + 1]
    o = s[:, 0:NOUT] / (den + 1e-16) + b2_ref[...]
    m = jnp.max(o, axis=1, keepdims=True)
    l = o - m
    out_ref[...] = l - jnp.log(jnp.sum(jnp.exp(l), axis=1, keepdims=True))


def _tc3(num2, b2):
    rb = 1000
    return pl.pallas_call(
        _tc3_body,
        grid=(NN // rb,),
        in_specs=[
            pl.BlockSpec((2, rb, C2), lambda nb: (0, nb, 0)),
            pl.BlockSpec((1, NOUT), lambda nb: (0, 0)),
        ],
        out_specs=pl.BlockSpec((rb, NOUT), lambda nb: (nb, 0)),
        out_shape=jax.ShapeDtypeStruct((NN, NOUT), _f32),
    )(num2, b2)


@jax.jit
def kernel(x, edge_index, W1, att_src1, att_dst1, b1,
           W2, att_src2, att_dst2, b2):
    src = edge_index[0]
    dst = edge_index[1]

    def _proj(att):
        ar = att.reshape(NQ, 2, CHID)
        s = jnp.zeros((NQ, 2, CHID, 2), _f32)
        s = s.at[:, 0, :, 0].set(ar[:, 0]).at[:, 1, :, 1].set(ar[:, 1])
        return s.reshape(NQ, 128, 2)

    s1 = _proj(att_src1)
    d1 = _proj(att_dst1)
    b1x = b1.reshape(NQ, 128)
    w2r = W2.reshape(NQ, 128, NOUT)
    sd2 = jnp.stack([att_src2[0], att_dst2[0]], axis=1)

    h1x, abig = _tc1(x, W1, s1, d1)
    num1 = _sc1(src, dst, h1x.reshape(NQ * NN, QC),
                abig.reshape(NQ * NN, 16))
    h2pad, a2 = _tc2(num1.reshape(NQ, NP, QC), b1x, w2r, sd2)
    num2 = _sc2(src, dst, h2pad, a2.reshape(-1))
    return _tc3(num2.reshape(2, NP, C2), b2.reshape(1, NOUT))
